# baseline (speedup 1.0000x reference)
import functools

import jax
import jax.numpy as jnp
from jax import lax
from jax.experimental import pallas as pl
from jax.experimental.pallas import tpu as pltpu
from jax.experimental.pallas import tpu_sc as plsc

_DIM = 64
_CTX = 4
_K = 6
_NC = 2
_NS = 16
_NW = _NC * _NS
_L = 16


def _transpose_body(wp_ref, out_ref):
    out_ref[...] = wp_ref[...].T


def _transpose_wp(WP, rows_out, cb):
    grid = rows_out // cb
    return pl.pallas_call(
        _transpose_body,
        grid=(grid,),
        in_specs=[pl.BlockSpec((_DIM, cb), lambda i: (0, i))],
        out_specs=pl.BlockSpec((cb, _DIM), lambda i: (i, 0)),
        out_shape=jax.ShapeDtypeStruct((rows_out, _DIM), jnp.float32),
    )(WP)


def _make_sc_call(B):
    bpw = B // _NW
    n_ctx_chunks = bpw * _CTX // 128
    n_tgt_chunks = bpw * _K // 128
    mesh = plsc.VectorSubcoreMesh(core_axis_name="c", subcore_axis_name="s")

    @functools.partial(
        pl.kernel,
        mesh=mesh,
        compiler_params=pltpu.CompilerParams(
            needs_layout_passes=False, use_tc_tiling_on_sc=False),
        out_type=jax.ShapeDtypeStruct((B * _K,), jnp.float32),
        scratch_types=[
            pltpu.VMEM((128,), jnp.int32),
            pltpu.VMEM((n_ctx_chunks * 128,), jnp.int32),
            pltpu.VMEM((n_tgt_chunks * 128,), jnp.int32),
            pltpu.VMEM((bpw, _DIM), jnp.float32),
            pltpu.VMEM((bpw * _CTX, _DIM), jnp.float32),
            pltpu.VMEM((bpw * _K, _DIM), jnp.float32),
            pltpu.VMEM((bpw * _K,), jnp.float32),
            pltpu.SemaphoreType.DMA,
        ],
    )
    def sc_kernel(ctx_hbm, doc_hbm, tgt_hbm, d_hbm, w_hbm, wpt_hbm, out_hbm,
                  doc_idx, ctx_idx, tgt_idx, d_rows, w_rows, wp_rows, out_v,
                  sem):
        wid = lax.axis_index("s") * _NC + lax.axis_index("c")
        base = wid * bpw

        pltpu.sync_copy(doc_hbm.at[pl.ds(base, bpw)], doc_idx)
        pltpu.sync_copy(
            ctx_hbm.at[pl.ds(wid * n_ctx_chunks * 128, n_ctx_chunks * 128)],
            ctx_idx)
        pltpu.sync_copy(
            tgt_hbm.at[pl.ds(wid * n_tgt_chunks * 128, n_tgt_chunks * 128)],
            tgt_idx)

        copies = [pltpu.async_copy(d_hbm.at[doc_idx], d_rows, sem)]
        for i in range(n_ctx_chunks):
            copies.append(pltpu.async_copy(
                w_hbm.at[ctx_idx.at[pl.ds(i * 128, 128)]],
                w_rows.at[pl.ds(i * 128, 128)], sem))
        for i in range(n_tgt_chunks):
            copies.append(pltpu.async_copy(
                wpt_hbm.at[tgt_idx.at[pl.ds(i * 128, 128)]],
                wp_rows.at[pl.ds(i * 128, 128)], sem))
        for c in copies:
            c.wait()

        def hbody(b, carry):
            for j in range(_DIM // _L):
                h = d_rows[b, pl.ds(j * _L, _L)]
                for c in range(_CTX):
                    h = h + w_rows[b * _CTX + c, pl.ds(j * _L, _L)]
                d_rows[b, pl.ds(j * _L, _L)] = h
            return carry

        lax.fori_loop(0, bpw, hbody, 0)

        lanes = lax.iota(jnp.int32, _L)

        def obody(g, carry):
            rows = jnp.full((_L,), g * _L, jnp.int32) + lanes
            bs = lax.div(rows, jnp.full((_L,), _K, jnp.int32))
            acc = jnp.zeros((_L,), jnp.float32)
            for d in range(_DIM):
                dd = jnp.full((_L,), d, jnp.int32)
                hv = plsc.load_gather(d_rows, [bs, dd])
                wv = plsc.load_gather(wp_rows, [rows, dd])
                acc = acc + hv * wv
            out_v[pl.ds(g * _L, _L)] = acc
            return carry

        lax.fori_loop(0, bpw * _K // _L, obody, 0)

        pltpu.sync_copy(out_v, out_hbm.at[pl.ds(base * _K, bpw * _K)])

    return sc_kernel


def kernel(ctxs, docs, targets, D, W, WP):
    B = ctxs.shape[0]
    n_words = WP.shape[1]
    cb = 1024
    rows_out = ((n_words + cb - 1) // cb) * cb

    ctx_flat = ctxs.reshape(-1)
    tgt_flat = targets.reshape(-1)
    WPT = _transpose_wp(WP, rows_out, cb)
    out_flat = _make_sc_call(B)(ctx_flat, docs, tgt_flat, D, W, WPT)
    return out_flat.reshape(B, _K)
